# baseline (speedup 1.0000x reference)
import jax
import jax.numpy as jnp
from jax.experimental import pallas as pl

_L = 4
_E = 8
_D = 80
_BLK = 2048
_SBLK = 32768


def _is_int_0_8(v):
    vi = v.astype(jnp.int32)
    return (vi.astype(jnp.float32) == v) & (v >= 0.0) & (v <= 7.0)


def _dot_t(x, w):
    return jax.lax.dot_general(
        x, w, dimension_numbers=(((1,), (1,)), ((), ())),
        preferred_element_type=jnp.float32)


def _top2_coeffs(g):
    T = g.shape[0]
    ids = jax.lax.broadcasted_iota(jnp.int32, (T, _E), 1)
    idf = ids.astype(jnp.float32)

    v0 = jnp.max(g, axis=1, keepdims=True)
    idx0 = jnp.min(jnp.where(g == v0, ids, _E), axis=1, keepdims=True)
    g1 = jnp.where(ids == idx0, -jnp.inf, g)
    v1 = jnp.max(g1, axis=1, keepdims=True)
    idx1 = jnp.min(jnp.where(g1 == v1, ids, _E), axis=1, keepdims=True)

    idx0f = idx0.astype(jnp.float32)
    idx1f = idx1.astype(jnp.float32)
    return (jnp.where(v0 == idf, idx0f, 0.0)
            + jnp.where(v1 == idf, idx1f, 0.0))


def _screen_body(x_ref, wg0_ref, bg_ref, flags_ref):
    g1 = _dot_t(x_ref[...], wg0_ref[...]) + bg_ref[0][None, :]
    hit = jnp.any(_is_int_0_8(g1)) | jnp.any(_is_int_0_8(bg_ref[...]))

    @pl.when(pl.program_id(0) == 0)
    def _init():
        flags_ref[...] = jnp.zeros(flags_ref.shape, jnp.float32)

    @pl.when(hit)
    def _mark():
        flags_ref[...] = jnp.ones(flags_ref.shape, jnp.float32)


def _full_body(x_ref, wg_ref, bg_ref, we_ref, be_ref, o_ref):
    T = x_ref.shape[0]
    xl = x_ref[...]
    for layer in range(_L):
        g = _dot_t(xl, wg_ref[layer]) + bg_ref[layer][None, :]
        c = _top2_coeffs(g)
        acc = jnp.zeros((T, _D), jnp.float32)
        for e in range(_E):
            eo = _dot_t(xl, we_ref[e]) + be_ref[e][None, :]
            acc = acc + c[:, e:e + 1] * eo
        xl = acc
    o_ref[...] = xl


def kernel(input_features, Wg, bg, We, be, interpret=False):
    B, S, D = input_features.shape
    N = B * S
    x = input_features.reshape(N, D)

    flags = pl.pallas_call(
        _screen_body,
        grid=(N // _SBLK,),
        in_specs=[
            pl.BlockSpec((_SBLK, D), lambda i: (i, 0)),
            pl.BlockSpec((_E, D), lambda i: (0, 0)),
            pl.BlockSpec((_L, _E), lambda i: (0, 0)),
        ],
        out_specs=pl.BlockSpec((8, 128), lambda i: (0, 0)),
        out_shape=jax.ShapeDtypeStruct((8, 128), jnp.float32),
        interpret=interpret,
    )(x, Wg[0], bg)

    def _full_path():
        return pl.pallas_call(
            _full_body,
            grid=(N // _BLK,),
            in_specs=[
                pl.BlockSpec((_BLK, D), lambda i: (i, 0)),
                pl.BlockSpec((_L, _E, D), lambda i: (0, 0, 0)),
                pl.BlockSpec((_L, _E), lambda i: (0, 0)),
                pl.BlockSpec((_E, D, D), lambda i: (0, 0, 0)),
                pl.BlockSpec((_E, D), lambda i: (0, 0)),
            ],
            out_specs=pl.BlockSpec((_BLK, D), lambda i: (i, 0)),
            out_shape=jax.ShapeDtypeStruct((N, D), jnp.float32),
            interpret=interpret,
        )(x, Wg, bg, We, be)

    out = jax.lax.cond(flags[0, 0] > 0.0, _full_path,
                       lambda: jnp.zeros((N, D), jnp.float32))
    return out.reshape(B, S, D)


# speedup vs baseline: 1.0391x; 1.0391x over previous
import jax
import jax.numpy as jnp
from jax.experimental import pallas as pl

_L = 4
_E = 8
_D = 80
_BLK = 2048
_SBLK = 16384


def _is_int_0_8(v):
    vi = v.astype(jnp.int32)
    return (vi.astype(jnp.float32) == v) & (v >= 0.0) & (v <= 7.0)


def _dot_t(x, w):
    return jax.lax.dot_general(
        x, w, dimension_numbers=(((1,), (1,)), ((), ())),
        preferred_element_type=jnp.float32)


def _top2_coeffs(g):
    T = g.shape[0]
    ids = jax.lax.broadcasted_iota(jnp.int32, (T, _E), 1)
    idf = ids.astype(jnp.float32)

    v0 = jnp.max(g, axis=1, keepdims=True)
    idx0 = jnp.min(jnp.where(g == v0, ids, _E), axis=1, keepdims=True)
    g1 = jnp.where(ids == idx0, -jnp.inf, g)
    v1 = jnp.max(g1, axis=1, keepdims=True)
    idx1 = jnp.min(jnp.where(g1 == v1, ids, _E), axis=1, keepdims=True)

    idx0f = idx0.astype(jnp.float32)
    idx1f = idx1.astype(jnp.float32)
    return (jnp.where(v0 == idf, idx0f, 0.0)
            + jnp.where(v1 == idf, idx1f, 0.0))


def _screen_body(x_ref, wg0_ref, bg_ref, flags_ref):
    g1 = _dot_t(x_ref[...], wg0_ref[...]) + bg_ref[0][None, :]
    hit = jnp.any(_is_int_0_8(g1)) | jnp.any(_is_int_0_8(bg_ref[...]))

    @pl.when(pl.program_id(0) == 0)
    def _init():
        flags_ref[...] = jnp.zeros(flags_ref.shape, jnp.float32)

    @pl.when(hit)
    def _mark():
        flags_ref[...] = jnp.ones(flags_ref.shape, jnp.float32)


def _full_body(x_ref, wg_ref, bg_ref, we_ref, be_ref, o_ref):
    T = x_ref.shape[0]
    xl = x_ref[...]
    for layer in range(_L):
        g = _dot_t(xl, wg_ref[layer]) + bg_ref[layer][None, :]
        c = _top2_coeffs(g)
        acc = jnp.zeros((T, _D), jnp.float32)
        for e in range(_E):
            eo = _dot_t(xl, we_ref[e]) + be_ref[e][None, :]
            acc = acc + c[:, e:e + 1] * eo
        xl = acc
    o_ref[...] = xl


def kernel(input_features, Wg, bg, We, be, interpret=False):
    B, S, D = input_features.shape
    N = B * S
    x = input_features.reshape(N, D)

    flags = pl.pallas_call(
        _screen_body,
        grid=(N // _SBLK,),
        in_specs=[
            pl.BlockSpec((_SBLK, D), lambda i: (i, 0)),
            pl.BlockSpec((_E, D), lambda i: (0, 0)),
            pl.BlockSpec((_L, _E), lambda i: (0, 0)),
        ],
        out_specs=pl.BlockSpec((8, 128), lambda i: (0, 0)),
        out_shape=jax.ShapeDtypeStruct((8, 128), jnp.float32),
        interpret=interpret,
    )(x, Wg[0], bg)

    def _full_path():
        return pl.pallas_call(
            _full_body,
            grid=(N // _BLK,),
            in_specs=[
                pl.BlockSpec((_BLK, D), lambda i: (i, 0)),
                pl.BlockSpec((_L, _E, D), lambda i: (0, 0, 0)),
                pl.BlockSpec((_L, _E), lambda i: (0, 0)),
                pl.BlockSpec((_E, D, D), lambda i: (0, 0, 0)),
                pl.BlockSpec((_E, D), lambda i: (0, 0)),
            ],
            out_specs=pl.BlockSpec((_BLK, D), lambda i: (i, 0)),
            out_shape=jax.ShapeDtypeStruct((N, D), jnp.float32),
            interpret=interpret,
        )(x, Wg, bg, We, be)

    out = jax.lax.cond(flags[0, 0] > 0.0, _full_path,
                       lambda: jnp.zeros((N, D), jnp.float32))
    return out.reshape(B, S, D)


# speedup vs baseline: 1.0488x; 1.0093x over previous
import jax
import jax.numpy as jnp
from jax.experimental import pallas as pl

_L = 4
_E = 8
_D = 80
_BLK = 2048
_SBLK = 8192


def _is_int_0_8(v):
    vi = v.astype(jnp.int32)
    return (vi.astype(jnp.float32) == v) & (v >= 0.0) & (v <= 7.0)


def _dot_t(x, w):
    return jax.lax.dot_general(
        x, w, dimension_numbers=(((1,), (1,)), ((), ())),
        preferred_element_type=jnp.float32)


def _top2_coeffs(g):
    T = g.shape[0]
    ids = jax.lax.broadcasted_iota(jnp.int32, (T, _E), 1)
    idf = ids.astype(jnp.float32)

    v0 = jnp.max(g, axis=1, keepdims=True)
    idx0 = jnp.min(jnp.where(g == v0, ids, _E), axis=1, keepdims=True)
    g1 = jnp.where(ids == idx0, -jnp.inf, g)
    v1 = jnp.max(g1, axis=1, keepdims=True)
    idx1 = jnp.min(jnp.where(g1 == v1, ids, _E), axis=1, keepdims=True)

    idx0f = idx0.astype(jnp.float32)
    idx1f = idx1.astype(jnp.float32)
    return (jnp.where(v0 == idf, idx0f, 0.0)
            + jnp.where(v1 == idf, idx1f, 0.0))


def _screen_body(x_ref, wg0_ref, bg_ref, flags_ref):
    g1 = _dot_t(x_ref[...], wg0_ref[...]) + bg_ref[0][None, :]
    hit = jnp.any(_is_int_0_8(g1)) | jnp.any(_is_int_0_8(bg_ref[...]))

    @pl.when(pl.program_id(0) == 0)
    def _init():
        flags_ref[...] = jnp.zeros(flags_ref.shape, jnp.float32)

    @pl.when(hit)
    def _mark():
        flags_ref[...] = jnp.ones(flags_ref.shape, jnp.float32)


def _full_body(x_ref, wg_ref, bg_ref, we_ref, be_ref, o_ref):
    T = x_ref.shape[0]
    xl = x_ref[...]
    for layer in range(_L):
        g = _dot_t(xl, wg_ref[layer]) + bg_ref[layer][None, :]
        c = _top2_coeffs(g)
        acc = jnp.zeros((T, _D), jnp.float32)
        for e in range(_E):
            eo = _dot_t(xl, we_ref[e]) + be_ref[e][None, :]
            acc = acc + c[:, e:e + 1] * eo
        xl = acc
    o_ref[...] = xl


def kernel(input_features, Wg, bg, We, be, interpret=False):
    B, S, D = input_features.shape
    N = B * S
    x = input_features.reshape(N, D)

    flags = pl.pallas_call(
        _screen_body,
        grid=(N // _SBLK,),
        in_specs=[
            pl.BlockSpec((_SBLK, D), lambda i: (i, 0)),
            pl.BlockSpec((_E, D), lambda i: (0, 0)),
            pl.BlockSpec((_L, _E), lambda i: (0, 0)),
        ],
        out_specs=pl.BlockSpec((8, 128), lambda i: (0, 0)),
        out_shape=jax.ShapeDtypeStruct((8, 128), jnp.float32),
        interpret=interpret,
    )(x, Wg[0], bg)

    def _full_path():
        return pl.pallas_call(
            _full_body,
            grid=(N // _BLK,),
            in_specs=[
                pl.BlockSpec((_BLK, D), lambda i: (i, 0)),
                pl.BlockSpec((_L, _E, D), lambda i: (0, 0, 0)),
                pl.BlockSpec((_L, _E), lambda i: (0, 0)),
                pl.BlockSpec((_E, D, D), lambda i: (0, 0, 0)),
                pl.BlockSpec((_E, D), lambda i: (0, 0)),
            ],
            out_specs=pl.BlockSpec((_BLK, D), lambda i: (i, 0)),
            out_shape=jax.ShapeDtypeStruct((N, D), jnp.float32),
            interpret=interpret,
        )(x, Wg, bg, We, be)

    out = jax.lax.cond(flags[0, 0] > 0.0, _full_path,
                       lambda: jnp.zeros((N, D), jnp.float32))
    return out.reshape(B, S, D)
